# baseline (speedup 1.0000x reference)
import functools

import jax
import jax.numpy as jnp
from jax import lax
from jax.experimental import pallas as pl
from jax.experimental.pallas import tpu as pltpu
from jax.experimental.pallas import tpu_sc as plsc

N = 10000
E = 160000
D = 256
DE = 16
HALF = 128
NC = 2
NS = 16
NW = NC * NS

C1 = 128
EP1 = 10240
K1 = EP1 // C1
NB1 = 2
NQ1 = 4

EPT = E // NW
C2 = 40
K2 = EPT // C2

NP = 10240
RPT = NP // NS
SR = 64
RB = 1000

_MESH = plsc.VectorSubcoreMesh(core_axis_name="c", subcore_axis_name="s")


def _sc_segment_sum_x(x2, src3, dst3, z_s):
  @functools.partial(
      pl.kernel,
      mesh=_MESH,
      out_type=jax.ShapeDtypeStruct((NC, NP, HALF), jnp.float32),
      scratch_types=[
          pltpu.VMEM((K1, C1), jnp.int32),
          pltpu.VMEM_SHARED((NP, HALF), jnp.float32),
      ] + [pltpu.VMEM((C1,), jnp.int32) for _ in range(NQ1)]
        + [pltpu.VMEM((C1, HALF), jnp.float32) for _ in range(NB1)]
        + [pltpu.SemaphoreType.DMA for _ in range(NQ1 + NB1)],
  )
  def k(x2_hbm, src_hbm, dst_hbm, zs_hbm, s_out, dst_all, s_sh, *rest):
    idxq = rest[:NQ1]
    bufs = rest[NQ1:NQ1 + NB1]
    isems = rest[NQ1 + NB1:2 * NQ1 + NB1]
    gsems = rest[2 * NQ1 + NB1:]
    c = lax.axis_index("c")
    s = lax.axis_index("s")
    r0 = s * RPT

    def load_idx(k_, q):
      pltpu.async_copy(src_hbm.at[s].at[pl.ds(k_ * C1, C1)],
                       idxq[q], isems[q])

    def wait_idx(k_, q):
      pltpu.make_async_copy(src_hbm.at[s].at[pl.ds(k_ * C1, C1)],
                            idxq[q], isems[q]).wait()

    def start_gather(k_, q, b):
      pltpu.async_copy(x2_hbm.at[c].at[idxq[q]], bufs[b], gsems[b])

    def wait_gather(k_, q, b):
      pltpu.make_async_copy(x2_hbm.at[c].at[idxq[q]], bufs[b],
                            gsems[b]).wait()

    pltpu.sync_copy(dst_hbm.at[s], dst_all)

    pltpu.sync_copy(zs_hbm, bufs[0])
    for j in range(RPT // C1):
      pltpu.sync_copy(bufs[0], s_sh.at[pl.ds(r0 + j * C1, C1)])
    plsc.subcore_barrier()

    for q in range(NQ1):
      load_idx(q, q)
    for b in range(NB1):
      wait_idx(b, b)
      start_gather(b, b, b)

    def outer(kk, carry):
      for u in range(NQ1):
        k_ = kk * NQ1 + u
        wait_gather(k_, u, u % NB1)
        pltpu.sync_copy(bufs[u % NB1], s_sh.at[dst_all.at[k_]], add=True)

        @pl.when(k_ + NQ1 < K1)
        def _():
          load_idx(k_ + NQ1, u)

        @pl.when(k_ + NB1 < K1)
        def _():
          wait_idx(k_ + NB1, (u + NB1) % NQ1)
          start_gather(k_ + NB1, (u + NB1) % NQ1, u % NB1)

      return carry

    lax.fori_loop(0, K1 // NQ1, outer, 0)
    plsc.subcore_barrier()

    for j in range(RPT // C1):
      rj = r0 + j * C1
      pltpu.sync_copy(s_sh.at[pl.ds(rj, C1)], bufs[0])
      pltpu.sync_copy(bufs[0], s_out.at[c].at[pl.ds(rj, C1)])

  return k(x2, src3, dst3, z_s)


def _sc_segment_sum_ea(ea3, dst3, z_s):
  @functools.partial(
      pl.kernel,
      mesh=_MESH,
      out_type=jax.ShapeDtypeStruct((NC, NP, HALF), jnp.float32),
      scratch_types=[
          pltpu.VMEM((K2, C2), jnp.int32),
          pltpu.VMEM((C2, DE), jnp.float32),
          pltpu.VMEM((C2, DE), jnp.float32),
          pltpu.VMEM((C2, HALF), jnp.float32),
          pltpu.VMEM_SHARED((NP, HALF), jnp.float32),
          pltpu.SemaphoreType.DMA,
          pltpu.SemaphoreType.DMA,
      ],
  )
  def k(ea_hbm, dst_hbm, zs_hbm, t_out,
        dst_all, ea_a, ea_b, rows_v, t_sh, sem_a, sem_b):
    c = lax.axis_index("c")
    s = lax.axis_index("s")
    w = c * NS + s
    r0 = s * RPT
    ea_bufs = (ea_a, ea_b)
    sems = (sem_a, sem_b)

    pltpu.sync_copy(dst_hbm.at[w], dst_all)
    pltpu.sync_copy(zs_hbm.at[pl.ds(0, C2)], rows_v)
    for j in range(RPT // C2):
      pltpu.sync_copy(rows_v, t_sh.at[pl.ds(r0 + j * C2, C2)])
    plsc.subcore_barrier()

    for b in range(2):
      pltpu.async_copy(ea_hbm.at[w].at[pl.ds(b * C2, C2)],
                       ea_bufs[b], sems[b])

    def step(k_, b):
      pltpu.make_async_copy(ea_hbm.at[w].at[pl.ds(k_ * C2, C2)],
                            ea_bufs[b], sems[b]).wait()

      def expand(j, carry2):
        rows_v[j, pl.ds(0, DE)] = ea_bufs[b][j]
        return carry2

      lax.fori_loop(0, C2, expand, 0)
      pltpu.sync_copy(rows_v, t_sh.at[dst_all.at[k_]], add=True)

    def outer(kk, carry):
      for b in range(2):
        k_ = kk * 2 + b
        step(k_, b)

        @pl.when(k_ + 2 < K2)
        def _():
          pltpu.async_copy(ea_hbm.at[w].at[pl.ds((k_ + 2) * C2, C2)],
                           ea_bufs[b], sems[b])

      return carry

    lax.fori_loop(0, K2 // 2, outer, 0)
    for k_ in range(2 * (K2 // 2), K2):
      step(k_, k_ % 2)
    plsc.subcore_barrier()

    for j in range(RPT // C2):
      rj = r0 + j * C2
      pltpu.sync_copy(t_sh.at[pl.ds(rj, C2)], rows_v)
      pltpu.sync_copy(rows_v, t_out.at[c].at[pl.ds(rj, C2)])

  return k(ea3, dst3, z_s)


def _tc_body(x_ref, s_ref, t_ref, w_ref, ws_ref, we_ref, b_ref, g_ref,
             be_ref, o_ref):
  w = w_ref[...]
  out = jnp.dot(s_ref[0], w[:HALF, :], preferred_element_type=jnp.float32)
  out += jnp.dot(s_ref[1], w[HALF:, :], preferred_element_type=jnp.float32)
  t = t_ref[0] + t_ref[1]
  out += jnp.dot(t[:, :DE], we_ref[...], preferred_element_type=jnp.float32)
  out += jnp.dot(x_ref[...], ws_ref[...], preferred_element_type=jnp.float32)
  out += b_ref[...]
  mu = jnp.mean(out, axis=-1, keepdims=True)
  var = jnp.mean(jnp.square(out - mu), axis=-1, keepdims=True)
  y = (out - mu) * lax.rsqrt(var + 1e-5) * g_ref[...] + be_ref[...]
  o_ref[...] = jnp.maximum(y, 0.0)


def _tc_combine(x, s_acc, t_acc, w, w_self, we, b, gamma, beta):
  return pl.pallas_call(
      _tc_body,
      grid=(N // RB,),
      in_specs=[
          pl.BlockSpec((RB, D), lambda i: (i, 0)),
          pl.BlockSpec((NC, RB, HALF), lambda i: (0, i, 0)),
          pl.BlockSpec((NC, RB, HALF), lambda i: (0, i, 0)),
          pl.BlockSpec((D, D), lambda i: (0, 0)),
          pl.BlockSpec((D, D), lambda i: (0, 0)),
          pl.BlockSpec((DE, D), lambda i: (0, 0)),
          pl.BlockSpec((1, D), lambda i: (0, 0)),
          pl.BlockSpec((1, D), lambda i: (0, 0)),
          pl.BlockSpec((1, D), lambda i: (0, 0)),
      ],
      out_specs=pl.BlockSpec((RB, D), lambda i: (i, 0)),
      out_shape=jax.ShapeDtypeStruct((N, D), jnp.float32),
  )(x, s_acc, t_acc, w, w_self, we, b.reshape(1, D), gamma.reshape(1, D),
    beta.reshape(1, D))


@jax.jit
def kernel(x, edge_index, edge_attr, W, W_self, We, b, gamma, beta):
  src = edge_index[0]
  dst = edge_index[1]
  x2 = x.reshape(N, NC, HALF).transpose(1, 0, 2)
  z_s = jnp.zeros((HALF, HALF), jnp.float32)

  src3 = jnp.pad(src.reshape(NS, E // NS), ((0, 0), (0, EP1 - E // NS)))
  dst3 = jnp.pad(dst.reshape(NS, E // NS), ((0, 0), (0, EP1 - E // NS)),
                 constant_values=NP - 1).reshape(NS, K1, C1)
  s_acc = _sc_segment_sum_x(x2, src3, dst3, z_s)

  dst_t = dst.reshape(NW, K2, C2)
  ea_t = edge_attr.reshape(NW, EPT, DE)
  t_acc = _sc_segment_sum_ea(ea_t, dst_t, z_s)

  return _tc_combine(x, s_acc, t_acc, W, W_self, We, b, gamma, beta)


# speedup vs baseline: 1.4364x; 1.4364x over previous
import functools

import jax
import jax.numpy as jnp
from jax import lax
from jax.experimental import pallas as pl
from jax.experimental.pallas import tpu as pltpu
from jax.experimental.pallas import tpu_sc as plsc

N = 10000
E = 160000
D = 256
DE = 16
HALF = 128
NC = 2
NS = 16
NW = NC * NS

C1 = 80
K1 = E // NS // C1
NB1 = 2

EPT = E // NW
C2 = 40
K2 = EPT // C2

NP = 10240
RPT = NP // NS
SR = 64
RB = 1000

_MESH = plsc.VectorSubcoreMesh(core_axis_name="c", subcore_axis_name="s")


def _sc_segment_sum_x(x2, src3, dst3, z_s):
  @functools.partial(
      pl.kernel,
      mesh=_MESH,
      out_type=jax.ShapeDtypeStruct((NC, NP, HALF), jnp.float32),
      scratch_types=[
          pltpu.VMEM((E // NS,), jnp.int32),
          pltpu.VMEM((K1, C1), jnp.int32),
          pltpu.VMEM_SHARED((NP, HALF), jnp.float32),
      ] + [pltpu.VMEM((C1, HALF), jnp.float32) for _ in range(NB1)]
        + [pltpu.SemaphoreType.DMA for _ in range(NB1)],
  )
  def k(x2_hbm, src_hbm, dst_hbm, zs_hbm, s_out,
        src_all, dst_all, s_sh, *bufs_and_sems):
    bufs = bufs_and_sems[:NB1]
    sems = bufs_and_sems[NB1:]
    c = lax.axis_index("c")
    s = lax.axis_index("s")
    r0 = s * RPT

    pltpu.sync_copy(src_hbm.at[s], src_all)
    pltpu.sync_copy(dst_hbm.at[s], dst_all)

    pltpu.sync_copy(zs_hbm.at[pl.ds(0, C1)], bufs[0])
    for j in range(RPT // C1):
      pltpu.sync_copy(bufs[0], s_sh.at[pl.ds(r0 + j * C1, C1)])
    plsc.subcore_barrier()

    def src_idx(k_):
      return src_all.at[pl.ds(k_ * C1, C1)]

    for b in range(NB1):
      pltpu.async_copy(x2_hbm.at[c].at[src_idx(b)], bufs[b], sems[b])

    def outer(kk, carry):
      for b in range(NB1):
        k_ = kk * NB1 + b
        pltpu.make_async_copy(
            x2_hbm.at[c].at[src_idx(k_)], bufs[b], sems[b]).wait()
        pltpu.sync_copy(bufs[b], s_sh.at[dst_all.at[k_]], add=True)

        @pl.when(k_ + NB1 < K1)
        def _():
          pltpu.async_copy(
              x2_hbm.at[c].at[src_idx(k_ + NB1)], bufs[b], sems[b])

      return carry

    lax.fori_loop(0, K1 // NB1, outer, 0)
    for k_ in range(NB1 * (K1 // NB1), K1):
      b = k_ % NB1
      pltpu.make_async_copy(
          x2_hbm.at[c].at[src_idx(k_)], bufs[b], sems[b]).wait()
      pltpu.sync_copy(bufs[b], s_sh.at[dst_all.at[k_]], add=True)
    plsc.subcore_barrier()

    for j in range(RPT // C1):
      rj = r0 + j * C1
      pltpu.sync_copy(s_sh.at[pl.ds(rj, C1)], bufs[0])
      pltpu.sync_copy(bufs[0], s_out.at[c].at[pl.ds(rj, C1)])

  return k(x2, src3, dst3, z_s)


def _sc_segment_sum_ea(ea3, dst3, z_s):
  @functools.partial(
      pl.kernel,
      mesh=_MESH,
      out_type=jax.ShapeDtypeStruct((NC, NP, HALF), jnp.float32),
      scratch_types=[
          pltpu.VMEM((K2, C2), jnp.int32),
          pltpu.VMEM((C2, DE), jnp.float32),
          pltpu.VMEM((C2, DE), jnp.float32),
          pltpu.VMEM((C2, HALF), jnp.float32),
          pltpu.VMEM_SHARED((NP, HALF), jnp.float32),
          pltpu.SemaphoreType.DMA,
          pltpu.SemaphoreType.DMA,
      ],
  )
  def k(ea_hbm, dst_hbm, zs_hbm, t_out,
        dst_all, ea_a, ea_b, rows_v, t_sh, sem_a, sem_b):
    c = lax.axis_index("c")
    s = lax.axis_index("s")
    w = c * NS + s
    r0 = s * RPT
    ea_bufs = (ea_a, ea_b)
    sems = (sem_a, sem_b)

    pltpu.sync_copy(dst_hbm.at[w], dst_all)
    pltpu.sync_copy(zs_hbm.at[pl.ds(0, C2)], rows_v)
    for j in range(RPT // C2):
      pltpu.sync_copy(rows_v, t_sh.at[pl.ds(r0 + j * C2, C2)])
    plsc.subcore_barrier()

    for b in range(2):
      pltpu.async_copy(ea_hbm.at[w].at[pl.ds(b * C2, C2)],
                       ea_bufs[b], sems[b])

    def step(k_, b):
      pltpu.make_async_copy(ea_hbm.at[w].at[pl.ds(k_ * C2, C2)],
                            ea_bufs[b], sems[b]).wait()

      def expand(j, carry2):
        rows_v[j, pl.ds(0, DE)] = ea_bufs[b][j]
        return carry2

      lax.fori_loop(0, C2, expand, 0)
      pltpu.sync_copy(rows_v, t_sh.at[dst_all.at[k_]], add=True)

    def outer(kk, carry):
      for b in range(2):
        k_ = kk * 2 + b
        step(k_, b)

        @pl.when(k_ + 2 < K2)
        def _():
          pltpu.async_copy(ea_hbm.at[w].at[pl.ds((k_ + 2) * C2, C2)],
                           ea_bufs[b], sems[b])

      return carry

    lax.fori_loop(0, K2 // 2, outer, 0)
    for k_ in range(2 * (K2 // 2), K2):
      step(k_, k_ % 2)
    plsc.subcore_barrier()

    for j in range(RPT // C2):
      rj = r0 + j * C2
      pltpu.sync_copy(t_sh.at[pl.ds(rj, C2)], rows_v)
      pltpu.sync_copy(rows_v, t_out.at[c].at[pl.ds(rj, C2)])

  return k(ea3, dst3, z_s)


def _tc_body(x_ref, s_ref, t_ref, w_ref, ws_ref, we_ref, b_ref, g_ref,
             be_ref, o_ref):
  w = w_ref[...]
  out = jnp.dot(s_ref[0], w[:HALF, :], preferred_element_type=jnp.float32)
  out += jnp.dot(s_ref[1], w[HALF:, :], preferred_element_type=jnp.float32)
  t = t_ref[0] + t_ref[1]
  out += jnp.dot(t[:, :DE], we_ref[...], preferred_element_type=jnp.float32)
  out += jnp.dot(x_ref[...], ws_ref[...], preferred_element_type=jnp.float32)
  out += b_ref[...]
  mu = jnp.mean(out, axis=-1, keepdims=True)
  var = jnp.mean(jnp.square(out - mu), axis=-1, keepdims=True)
  y = (out - mu) * lax.rsqrt(var + 1e-5) * g_ref[...] + be_ref[...]
  o_ref[...] = jnp.maximum(y, 0.0)


def _tc_combine(x, s_acc, t_acc, w, w_self, we, b, gamma, beta):
  return pl.pallas_call(
      _tc_body,
      grid=(N // RB,),
      in_specs=[
          pl.BlockSpec((RB, D), lambda i: (i, 0)),
          pl.BlockSpec((NC, RB, HALF), lambda i: (0, i, 0)),
          pl.BlockSpec((NC, RB, HALF), lambda i: (0, i, 0)),
          pl.BlockSpec((D, D), lambda i: (0, 0)),
          pl.BlockSpec((D, D), lambda i: (0, 0)),
          pl.BlockSpec((DE, D), lambda i: (0, 0)),
          pl.BlockSpec((1, D), lambda i: (0, 0)),
          pl.BlockSpec((1, D), lambda i: (0, 0)),
          pl.BlockSpec((1, D), lambda i: (0, 0)),
      ],
      out_specs=pl.BlockSpec((RB, D), lambda i: (i, 0)),
      out_shape=jax.ShapeDtypeStruct((N, D), jnp.float32),
  )(x, s_acc, t_acc, w, w_self, we, b.reshape(1, D), gamma.reshape(1, D),
    beta.reshape(1, D))


@jax.jit
def kernel(x, edge_index, edge_attr, W, W_self, We, b, gamma, beta):
  src = edge_index[0]
  dst = edge_index[1]
  x2 = x.reshape(N, NC, HALF).transpose(1, 0, 2)
  z_s = jnp.zeros((HALF, HALF), jnp.float32)

  src3 = src.reshape(NS, E // NS)
  dst3 = dst.reshape(NS, K1, C1)
  s_acc = _sc_segment_sum_x(x2, src3, dst3, z_s)

  dst_t = dst.reshape(NW, K2, C2)
  ea_t = edge_attr.reshape(NW, EPT, DE)
  t_acc = _sc_segment_sum_ea(ea_t, dst_t, z_s)

  return _tc_combine(x, s_acc, t_acc, W, W_self, We, b, gamma, beta)


# speedup vs baseline: 1.5845x; 1.1031x over previous
import functools

import jax
import jax.numpy as jnp
from jax import lax
from jax.experimental import pallas as pl
from jax.experimental.pallas import tpu as pltpu
from jax.experimental.pallas import tpu_sc as plsc

N = 10000
E = 160000
D = 256
DE = 16
HALF = 128
NC = 2
NS = 16
NW = NC * NS

C1 = 80
K1 = E // NS // C1
NB1 = 2

EPT = E // NW
C2 = 40
K2 = EPT // C2

NP = 10240
RPT = NP // NS
SR = 64
RB = 1000

_MESH = plsc.VectorSubcoreMesh(core_axis_name="c", subcore_axis_name="s")


def _sc_segment_sum_x(x2, src3, dst3, z_s):
  @functools.partial(
      pl.kernel,
      mesh=_MESH,
      out_type=jax.ShapeDtypeStruct((NC, NP, HALF), jnp.float32),
      scratch_types=[
          pltpu.VMEM((E // NS,), jnp.int32),
          pltpu.VMEM((K1, C1), jnp.int32),
          pltpu.VMEM_SHARED((NP, HALF), jnp.float32),
      ] + [pltpu.VMEM((C1,), jnp.int32) for _ in range(NB1)]
        + [pltpu.VMEM((C1, HALF), jnp.float32) for _ in range(NB1)]
        + [pltpu.SemaphoreType.DMA for _ in range(NB1)],
  )
  def k(xf_hbm, src_hbm, dst_hbm, zs_hbm, s_out,
        src_all, dst_all, s_sh, *rest):
    cidx = rest[:NB1]
    bufs = rest[NB1:2 * NB1]
    sems = rest[2 * NB1:]
    c = lax.axis_index("c")
    s = lax.axis_index("s")
    r0 = s * RPT

    pltpu.sync_copy(src_hbm.at[s], src_all)
    pltpu.sync_copy(dst_hbm.at[s], dst_all)

    pltpu.sync_copy(zs_hbm.at[pl.ds(0, C1)], bufs[0])
    for j in range(RPT // C1):
      pltpu.sync_copy(bufs[0], s_sh.at[pl.ds(r0 + j * C1, C1)])
    plsc.subcore_barrier()

    def comp_idx(k_, b):
      for g in range(C1 // 16):
        v = src_all[pl.ds(k_ * C1 + g * 16, 16)]
        cidx[b][pl.ds(g * 16, 16)] = v + v + c

    for b in range(NB1):
      comp_idx(b, b)
      pltpu.async_copy(xf_hbm.at[cidx[b]], bufs[b], sems[b])

    def outer(kk, carry):
      for b in range(NB1):
        k_ = kk * NB1 + b
        pltpu.make_async_copy(xf_hbm.at[cidx[b]], bufs[b], sems[b]).wait()
        pltpu.sync_copy(bufs[b], s_sh.at[dst_all.at[k_]], add=True)

        @pl.when(k_ + NB1 < K1)
        def _():
          comp_idx(k_ + NB1, b)
          pltpu.async_copy(xf_hbm.at[cidx[b]], bufs[b], sems[b])

      return carry

    lax.fori_loop(0, K1 // NB1, outer, 0)
    for k_ in range(NB1 * (K1 // NB1), K1):
      b = k_ % NB1
      pltpu.make_async_copy(xf_hbm.at[cidx[b]], bufs[b], sems[b]).wait()
      pltpu.sync_copy(bufs[b], s_sh.at[dst_all.at[k_]], add=True)
    plsc.subcore_barrier()

    for j in range(RPT // C1):
      rj = r0 + j * C1
      pltpu.sync_copy(s_sh.at[pl.ds(rj, C1)], bufs[0])
      pltpu.sync_copy(bufs[0], s_out.at[c].at[pl.ds(rj, C1)])

  return k(x2, src3, dst3, z_s)


def _sc_segment_sum_ea(ea3, dst3, z_s):
  @functools.partial(
      pl.kernel,
      mesh=_MESH,
      out_type=jax.ShapeDtypeStruct((NC, NP, HALF), jnp.float32),
      scratch_types=[
          pltpu.VMEM((K2, C2), jnp.int32),
          pltpu.VMEM((C2, DE), jnp.float32),
          pltpu.VMEM((C2, DE), jnp.float32),
          pltpu.VMEM((C2, HALF), jnp.float32),
          pltpu.VMEM_SHARED((NP, HALF), jnp.float32),
          pltpu.SemaphoreType.DMA,
          pltpu.SemaphoreType.DMA,
      ],
  )
  def k(ea_hbm, dst_hbm, zs_hbm, t_out,
        dst_all, ea_a, ea_b, rows_v, t_sh, sem_a, sem_b):
    c = lax.axis_index("c")
    s = lax.axis_index("s")
    w = c * NS + s
    r0 = s * RPT
    ea_bufs = (ea_a, ea_b)
    sems = (sem_a, sem_b)

    pltpu.sync_copy(dst_hbm.at[w], dst_all)
    pltpu.sync_copy(zs_hbm.at[pl.ds(0, C2)], rows_v)
    for j in range(RPT // C2):
      pltpu.sync_copy(rows_v, t_sh.at[pl.ds(r0 + j * C2, C2)])
    plsc.subcore_barrier()

    for b in range(2):
      pltpu.async_copy(ea_hbm.at[w].at[pl.ds(b * C2, C2)],
                       ea_bufs[b], sems[b])

    def step(k_, b):
      pltpu.make_async_copy(ea_hbm.at[w].at[pl.ds(k_ * C2, C2)],
                            ea_bufs[b], sems[b]).wait()

      for j in range(C2):
        rows_v[j, pl.ds(0, DE)] = ea_bufs[b][j]
      pltpu.sync_copy(rows_v, t_sh.at[dst_all.at[k_]], add=True)

    def outer(kk, carry):
      for b in range(2):
        k_ = kk * 2 + b
        step(k_, b)

        @pl.when(k_ + 2 < K2)
        def _():
          pltpu.async_copy(ea_hbm.at[w].at[pl.ds((k_ + 2) * C2, C2)],
                           ea_bufs[b], sems[b])

      return carry

    lax.fori_loop(0, K2 // 2, outer, 0)
    for k_ in range(2 * (K2 // 2), K2):
      step(k_, k_ % 2)
    plsc.subcore_barrier()

    for j in range(RPT // C2):
      rj = r0 + j * C2
      pltpu.sync_copy(t_sh.at[pl.ds(rj, C2)], rows_v)
      pltpu.sync_copy(rows_v, t_out.at[c].at[pl.ds(rj, C2)])

  return k(ea3, dst3, z_s)


def _tc_body(x_ref, s_ref, t_ref, w_ref, ws_ref, we_ref, b_ref, g_ref,
             be_ref, o_ref):
  w = w_ref[...]
  out = jnp.dot(s_ref[0], w[:HALF, :], preferred_element_type=jnp.float32)
  out += jnp.dot(s_ref[1], w[HALF:, :], preferred_element_type=jnp.float32)
  t = t_ref[0] + t_ref[1]
  out += jnp.dot(t[:, :DE], we_ref[...], preferred_element_type=jnp.float32)
  out += jnp.dot(x_ref[...], ws_ref[...], preferred_element_type=jnp.float32)
  out += b_ref[...]
  mu = jnp.mean(out, axis=-1, keepdims=True)
  var = jnp.mean(jnp.square(out - mu), axis=-1, keepdims=True)
  y = (out - mu) * lax.rsqrt(var + 1e-5) * g_ref[...] + be_ref[...]
  o_ref[...] = jnp.maximum(y, 0.0)


def _tc_combine(x, s_acc, t_acc, w, w_self, we, b, gamma, beta):
  return pl.pallas_call(
      _tc_body,
      grid=(N // RB,),
      in_specs=[
          pl.BlockSpec((RB, D), lambda i: (i, 0)),
          pl.BlockSpec((NC, RB, HALF), lambda i: (0, i, 0)),
          pl.BlockSpec((NC, RB, HALF), lambda i: (0, i, 0)),
          pl.BlockSpec((D, D), lambda i: (0, 0)),
          pl.BlockSpec((D, D), lambda i: (0, 0)),
          pl.BlockSpec((DE, D), lambda i: (0, 0)),
          pl.BlockSpec((1, D), lambda i: (0, 0)),
          pl.BlockSpec((1, D), lambda i: (0, 0)),
          pl.BlockSpec((1, D), lambda i: (0, 0)),
      ],
      out_specs=pl.BlockSpec((RB, D), lambda i: (i, 0)),
      out_shape=jax.ShapeDtypeStruct((N, D), jnp.float32),
  )(x, s_acc, t_acc, w, w_self, we, b.reshape(1, D), gamma.reshape(1, D),
    beta.reshape(1, D))


@jax.jit
def kernel(x, edge_index, edge_attr, W, W_self, We, b, gamma, beta):
  src = edge_index[0]
  dst = edge_index[1]
  xf = x.reshape(NC * N, HALF)
  z_s = jnp.zeros((HALF, HALF), jnp.float32)

  src3 = src.reshape(NS, E // NS)
  dst3 = dst.reshape(NS, K1, C1)
  s_acc = _sc_segment_sum_x(xf, src3, dst3, z_s)

  dst_t = dst.reshape(NW, K2, C2)
  ea_t = edge_attr.reshape(NW, EPT, DE)
  t_acc = _sc_segment_sum_ea(ea_t, dst_t, z_s)

  return _tc_combine(x, s_acc, t_acc, W, W_self, We, b, gamma, beta)


# speedup vs baseline: 1.6386x; 1.0341x over previous
import functools

import jax
import jax.numpy as jnp
from jax import lax
from jax.experimental import pallas as pl
from jax.experimental.pallas import tpu as pltpu
from jax.experimental.pallas import tpu_sc as plsc

N = 10000
E = 160000
D = 256
DE = 16
HALF = 128
NC = 2
NS = 16
NW = NC * NS

C1 = 80
K1 = E // NS // C1
NB1 = 2

EPT = E // NW
C2 = 40
K2 = EPT // C2

NP = 10240
RPT = NP // NS
SR = 64
RB = 1000

_MESH = plsc.VectorSubcoreMesh(core_axis_name="c", subcore_axis_name="s")


def _sc_segment_sum_x(x2, src3, dst3, z_s):
  @functools.partial(
      pl.kernel,
      mesh=_MESH,
      out_type=jax.ShapeDtypeStruct((NC, NP, HALF), jnp.float32),
      scratch_types=[
          pltpu.VMEM((E // NS,), jnp.int32),
          pltpu.VMEM((K1, C1), jnp.int32),
          pltpu.VMEM_SHARED((NP, HALF), jnp.float32),
      ] + [pltpu.VMEM((C1,), jnp.int32) for _ in range(NB1)]
        + [pltpu.VMEM((C1, HALF), jnp.float32) for _ in range(NB1)]
        + [pltpu.SemaphoreType.DMA for _ in range(NB1)],
  )
  def k(xf_hbm, src_hbm, dst_hbm, zs_hbm, s_out,
        src_all, dst_all, s_sh, *rest):
    cidx = rest[:NB1]
    bufs = rest[NB1:2 * NB1]
    sems = rest[2 * NB1:]
    c = lax.axis_index("c")
    s = lax.axis_index("s")
    r0 = s * RPT

    pltpu.sync_copy(src_hbm.at[s], src_all)
    pltpu.sync_copy(dst_hbm.at[s], dst_all)

    pltpu.sync_copy(zs_hbm.at[pl.ds(0, C1)], bufs[0])
    for j in range(RPT // C1):
      pltpu.sync_copy(bufs[0], s_sh.at[pl.ds(r0 + j * C1, C1)])
    plsc.subcore_barrier()

    def comp_idx(k_, b):
      for g in range(C1 // 16):
        v = src_all[pl.ds(k_ * C1 + g * 16, 16)]
        cidx[b][pl.ds(g * 16, 16)] = v + v + c

    for b in range(NB1):
      comp_idx(b, b)
      pltpu.async_copy(xf_hbm.at[cidx[b]], bufs[b], sems[b])

    def outer(kk, carry):
      for b in range(NB1):
        k_ = kk * NB1 + b
        pltpu.make_async_copy(xf_hbm.at[cidx[b]], bufs[b], sems[b]).wait()
        pltpu.sync_copy(bufs[b], s_sh.at[dst_all.at[k_]], add=True)

        @pl.when(k_ + NB1 < K1)
        def _():
          comp_idx(k_ + NB1, b)
          pltpu.async_copy(xf_hbm.at[cidx[b]], bufs[b], sems[b])

      return carry

    lax.fori_loop(0, K1 // NB1, outer, 0)
    for k_ in range(NB1 * (K1 // NB1), K1):
      b = k_ % NB1
      pltpu.make_async_copy(xf_hbm.at[cidx[b]], bufs[b], sems[b]).wait()
      pltpu.sync_copy(bufs[b], s_sh.at[dst_all.at[k_]], add=True)
    plsc.subcore_barrier()

    for j in range(RPT // C1):
      rj = r0 + j * C1
      pltpu.sync_copy(s_sh.at[pl.ds(rj, C1)], bufs[0])
      pltpu.sync_copy(bufs[0], s_out.at[c].at[pl.ds(rj, C1)])

  return k(x2, src3, dst3, z_s)


def _sc_segment_sum_ea(ea3, dst3, z_s):
  @functools.partial(
      pl.kernel,
      mesh=_MESH,
      out_type=jax.ShapeDtypeStruct((NC, NP, HALF), jnp.float32),
      scratch_types=[
          pltpu.VMEM((K2, C2), jnp.int32),
          pltpu.VMEM((C2, DE), jnp.float32),
          pltpu.VMEM((C2, DE), jnp.float32),
          pltpu.VMEM((C2, HALF), jnp.float32),
          pltpu.VMEM((C2, HALF), jnp.float32),
          pltpu.VMEM_SHARED((NP, HALF), jnp.float32),
          pltpu.SemaphoreType.DMA,
          pltpu.SemaphoreType.DMA,
          pltpu.SemaphoreType.DMA,
          pltpu.SemaphoreType.DMA,
      ],
  )
  def k(ea_hbm, dst_hbm, zs_hbm, t_out,
        dst_all, ea_a, ea_b, rows_a, rows_b, t_sh, sem_a, sem_b,
        ssem_a, ssem_b):
    c = lax.axis_index("c")
    s = lax.axis_index("s")
    w = c * NS + s
    r0 = s * RPT
    ea_bufs = (ea_a, ea_b)
    rows = (rows_a, rows_b)
    sems = (sem_a, sem_b)
    ssems = (ssem_a, ssem_b)

    pltpu.sync_copy(dst_hbm.at[w], dst_all)
    pltpu.sync_copy(zs_hbm.at[pl.ds(0, C2)], rows_a)
    pltpu.sync_copy(zs_hbm.at[pl.ds(0, C2)], rows_b)
    for j in range(RPT // C2):
      pltpu.sync_copy(rows_a, t_sh.at[pl.ds(r0 + j * C2, C2)])
    plsc.subcore_barrier()

    for b in range(2):
      pltpu.async_copy(ea_hbm.at[w].at[pl.ds(b * C2, C2)],
                       ea_bufs[b], sems[b])

    def outer(kk, carry):
      for b in range(2):
        k_ = kk * 2 + b
        pltpu.make_async_copy(ea_hbm.at[w].at[pl.ds(k_ * C2, C2)],
                              ea_bufs[b], sems[b]).wait()
        @pl.when(k_ >= 2)
        def _():
          pltpu.make_async_copy(rows[b], t_sh.at[dst_all.at[k_ - 2]],
                                ssems[b]).wait()

        for j in range(C2):
          rows[b][j, pl.ds(0, DE)] = ea_bufs[b][j]
        pltpu.async_copy(rows[b], t_sh.at[dst_all.at[k_]], ssems[b],
                         add=True)

        @pl.when(k_ + 2 < K2)
        def _():
          pltpu.async_copy(ea_hbm.at[w].at[pl.ds((k_ + 2) * C2, C2)],
                           ea_bufs[b], sems[b])

      return carry

    lax.fori_loop(0, K2 // 2, outer, 0)
    for k_ in range(2 * (K2 // 2), K2):
      b = k_ % 2
      pltpu.make_async_copy(ea_hbm.at[w].at[pl.ds(k_ * C2, C2)],
                            ea_bufs[b], sems[b]).wait()
      pltpu.make_async_copy(rows[b], t_sh.at[dst_all.at[k_ - 2]],
                            ssems[b]).wait()
      for j in range(C2):
        rows[b][j, pl.ds(0, DE)] = ea_bufs[b][j]
      pltpu.sync_copy(rows[b], t_sh.at[dst_all.at[k_]], add=True)
    for k_ in ([K2 - 2] if K2 % 2 else [K2 - 2, K2 - 1]):
      pltpu.make_async_copy(rows[k_ % 2], t_sh.at[dst_all.at[k_]],
                            ssems[k_ % 2]).wait()
    plsc.subcore_barrier()

    for j in range(RPT // C2):
      rj = r0 + j * C2
      pltpu.sync_copy(t_sh.at[pl.ds(rj, C2)], rows_a)
      pltpu.sync_copy(rows_a, t_out.at[c].at[pl.ds(rj, C2)])

  return k(ea3, dst3, z_s)


def _tc_body(x_ref, s_ref, t_ref, w_ref, ws_ref, we_ref, b_ref, g_ref,
             be_ref, o_ref):
  w = w_ref[...]
  out = jnp.dot(s_ref[0], w[:HALF, :], preferred_element_type=jnp.float32)
  out += jnp.dot(s_ref[1], w[HALF:, :], preferred_element_type=jnp.float32)
  t = t_ref[0] + t_ref[1]
  out += jnp.dot(t[:, :DE], we_ref[...], preferred_element_type=jnp.float32)
  out += jnp.dot(x_ref[...], ws_ref[...], preferred_element_type=jnp.float32)
  out += b_ref[...]
  mu = jnp.mean(out, axis=-1, keepdims=True)
  var = jnp.mean(jnp.square(out - mu), axis=-1, keepdims=True)
  y = (out - mu) * lax.rsqrt(var + 1e-5) * g_ref[...] + be_ref[...]
  o_ref[...] = jnp.maximum(y, 0.0)


def _tc_combine(x, s_acc, t_acc, w, w_self, we, b, gamma, beta):
  return pl.pallas_call(
      _tc_body,
      grid=(N // RB,),
      in_specs=[
          pl.BlockSpec((RB, D), lambda i: (i, 0)),
          pl.BlockSpec((NC, RB, HALF), lambda i: (0, i, 0)),
          pl.BlockSpec((NC, RB, HALF), lambda i: (0, i, 0)),
          pl.BlockSpec((D, D), lambda i: (0, 0)),
          pl.BlockSpec((D, D), lambda i: (0, 0)),
          pl.BlockSpec((DE, D), lambda i: (0, 0)),
          pl.BlockSpec((1, D), lambda i: (0, 0)),
          pl.BlockSpec((1, D), lambda i: (0, 0)),
          pl.BlockSpec((1, D), lambda i: (0, 0)),
      ],
      out_specs=pl.BlockSpec((RB, D), lambda i: (i, 0)),
      out_shape=jax.ShapeDtypeStruct((N, D), jnp.float32),
  )(x, s_acc, t_acc, w, w_self, we, b.reshape(1, D), gamma.reshape(1, D),
    beta.reshape(1, D))


@jax.jit
def kernel(x, edge_index, edge_attr, W, W_self, We, b, gamma, beta):
  src = edge_index[0]
  dst = edge_index[1]
  xf = x.reshape(NC * N, HALF)
  z_s = jnp.zeros((HALF, HALF), jnp.float32)

  src3 = src.reshape(NS, E // NS)
  dst3 = dst.reshape(NS, K1, C1)
  s_acc = _sc_segment_sum_x(xf, src3, dst3, z_s)

  dst_t = dst.reshape(NW, K2, C2)
  ea_t = edge_attr.reshape(NW, EPT, DE)
  t_acc = _sc_segment_sum_ea(ea_t, dst_t, z_s)

  return _tc_combine(x, s_acc, t_acc, W, W_self, We, b, gamma, beta)
